# baseline (speedup 1.0000x reference)
import jax
import jax.numpy as jnp
from jax.experimental import pallas as pl

_LAMDA = 0.9
_N = 16384
_C = 1000
_BR = 1024
_TW = 128
_NF = 7
_TAIL = _C - _TW


def _loss_step(tbl_ref, act_ref, delta_ref, prop_ref, out_ref):
    pid = pl.program_id(0)

    act = act_ref[...]
    shape = (_BR, _TW)
    a_lo = jnp.broadcast_to(act % _TW, shape)
    a_hi = jnp.broadcast_to(act // _TW, shape)
    a_tail = jnp.broadcast_to(jnp.clip(act - _TAIL, 0, _TW - 1), shape)

    acc = jnp.zeros(shape, jnp.float32)
    for j in range(_NF):
        tile = tbl_ref[:, j * _TW:(j + 1) * _TW]
        g = jnp.take_along_axis(tile, a_lo, axis=1)
        acc = acc + jnp.where(a_hi == j, g, 0.0)
    tail = tbl_ref[:, _TAIL:_C]
    g = jnp.take_along_axis(tail, a_tail, axis=1)
    acc = acc + jnp.where(a_hi == _NF, g, 0.0)

    w = (-delta_ref[...] - _LAMDA) / prop_ref[...]
    wb = jnp.broadcast_to(w, shape)
    partial = jnp.sum(acc * wb).reshape(1, 1) * (1.0 / (_N * _TW))

    @pl.when(pid == 0)
    def _init():
        out_ref[...] = jnp.zeros_like(out_ref)

    out_ref[...] += partial


@jax.jit
def kernel(output, action, delta, prop):
    act2 = action.astype(jnp.int32).reshape(_N, 1)
    delta2 = delta.reshape(_N, 1)
    prop2 = prop.reshape(_N, 1)
    out = pl.pallas_call(
        _loss_step,
        grid=(_N // _BR,),
        in_specs=[
            pl.BlockSpec((_BR, _C), lambda i: (i, 0)),
            pl.BlockSpec((_BR, 1), lambda i: (i, 0)),
            pl.BlockSpec((_BR, 1), lambda i: (i, 0)),
            pl.BlockSpec((_BR, 1), lambda i: (i, 0)),
        ],
        out_specs=pl.BlockSpec((1, 1), lambda i: (0, 0)),
        out_shape=jax.ShapeDtypeStruct((1, 1), jnp.float32),
    )(output, act2, delta2, prop2)
    return out[0, 0]


# speedup vs baseline: 1.6954x; 1.6954x over previous
import functools

import jax
import jax.numpy as jnp
from jax import lax
from jax.experimental import pallas as pl

_LAMDA = 0.9
_N = 16384
_C = 1000
_BR = 1024


def _loss_step(tbl_ref, act_ref, delta_ref, prop_ref, out_ref):
    pid = pl.program_id(0)

    tbl = tbl_ref[...]
    act = act_ref[...]
    cols = lax.broadcasted_iota(jnp.int32, (_BR, _C), 1)
    mask = cols == act[:, None]
    sel = jnp.sum(jnp.where(mask, tbl, 0.0), axis=1)
    w = (-delta_ref[...] - _LAMDA) / prop_ref[...]
    partial = jnp.sum(sel * w).reshape(1, 1) * (1.0 / _N)

    @pl.when(pid == 0)
    def _init():
        out_ref[...] = jnp.zeros_like(out_ref)

    out_ref[...] += partial


@jax.jit
def kernel(output, action, delta, prop):
    out = pl.pallas_call(
        _loss_step,
        grid=(_N // _BR,),
        in_specs=[
            pl.BlockSpec((_BR, _C), lambda i: (i, 0)),
            pl.BlockSpec((_BR,), lambda i: (i,)),
            pl.BlockSpec((_BR,), lambda i: (i,)),
            pl.BlockSpec((_BR,), lambda i: (i,)),
        ],
        out_specs=pl.BlockSpec((1, 1), lambda i: (0, 0)),
        out_shape=jax.ShapeDtypeStruct((1, 1), jnp.float32),
    )(output, action.astype(jnp.int32), delta, prop)
    return out[0, 0]
